# Initial kernel scaffold; baseline (speedup 1.0000x reference)
import jax
import jax.numpy as jnp
from jax.experimental import pallas as pl


def kernel(inputs, weight):
    raise NotImplementedError("write your pallas kernel here")


# speedup vs baseline: 2.6934x; 2.6934x over previous
import jax
import jax.numpy as jnp
from jax.experimental import pallas as pl
from jax.experimental.pallas import tpu as pltpu

B = 16
C = 512
E = 512
HW = 64 * 64
TN = 512
NT = HW // TN
LOSS_SCALE = 1.25 / (B * HW * C)


def _vq_body(x_ref, w_ref, q_ref, idx_ref, loss_ref, perp_ref, pres_ref):
    b = pl.program_id(0)
    t = pl.program_id(1)
    x = x_ref[0]
    w = w_ref[...]
    wsq = jnp.sum(w * w, axis=1, keepdims=True)
    xsq = jnp.sum(x * x, axis=0, keepdims=True)
    scores = (wsq + xsq) - 2.0 * jax.lax.dot_general(
        w, x, (((1,), (0,)), ((), ())), preferred_element_type=jnp.float32)
    m = jnp.min(scores, axis=0, keepdims=True)
    iota_e = jax.lax.broadcasted_iota(jnp.int32, (E, TN), 0)
    idx = jnp.min(jnp.where(scores == m, iota_e, E), axis=0, keepdims=True)
    idx_ref[0] = idx
    onehot = (iota_e == idx).astype(jnp.float32)
    q = jax.lax.dot_general(
        w, onehot, (((0,), (0,)), ((), ())), preferred_element_type=jnp.float32)
    q_ref[0] = q

    @pl.when(jnp.logical_and(b == 0, t == 0))
    def _init():
        loss_ref[...] = jnp.zeros((1, 1), jnp.float32)
        perp_ref[...] = jnp.zeros((1, 1), jnp.float32)

    loss_ref[...] += jnp.sum(m, axis=1, keepdims=True)

    contrib = jnp.max(onehot, axis=1, keepdims=True)

    @pl.when(t == 0)
    def _reset():
        pres_ref[...] = jnp.zeros_like(pres_ref)

    pres_ref[...] = jnp.maximum(pres_ref[...], contrib)

    @pl.when(t == NT - 1)
    def _batch_done():
        perp_ref[...] += jnp.sum(pres_ref[...], axis=0, keepdims=True)

    @pl.when(jnp.logical_and(b == B - 1, t == NT - 1))
    def _finish():
        loss_ref[...] *= LOSS_SCALE
        perp_ref[...] *= 1.0 / B


def kernel(inputs, weight):
    x3 = inputs.reshape(B, C, HW)
    q, idxr, loss, perp = pl.pallas_call(
        _vq_body,
        grid=(B, NT),
        in_specs=[
            pl.BlockSpec((1, C, TN), lambda b, t: (b, 0, t)),
            pl.BlockSpec((E, C), lambda b, t: (0, 0)),
        ],
        out_specs=[
            pl.BlockSpec((1, C, TN), lambda b, t: (b, 0, t)),
            pl.BlockSpec((1, 1, TN), lambda b, t: (b * NT + t, 0, 0)),
            pl.BlockSpec((1, 1), lambda b, t: (0, 0)),
            pl.BlockSpec((1, 1), lambda b, t: (0, 0)),
        ],
        out_shape=[
            jax.ShapeDtypeStruct((B, C, HW), jnp.float32),
            jax.ShapeDtypeStruct((B * NT, 1, TN), jnp.int32),
            jax.ShapeDtypeStruct((1, 1), jnp.float32),
            jax.ShapeDtypeStruct((1, 1), jnp.float32),
        ],
        scratch_shapes=[pltpu.VMEM((E, 1), jnp.float32)],
        compiler_params=pltpu.CompilerParams(
            dimension_semantics=("arbitrary", "arbitrary")),
    )(x3, weight)
    quantized_out = q.reshape(B, C, 64, 64)
    encoding_indices = idxr.reshape(B, HW)
    return (loss[0, 0], quantized_out, perp[0, 0], encoding_indices)


# speedup vs baseline: 3.3028x; 1.2262x over previous
import jax
import jax.numpy as jnp
from jax.experimental import pallas as pl
from jax.experimental.pallas import tpu as pltpu

B = 16
C = 512
E = 512
HW = 64 * 64
TN = 2048
NT = HW // TN
LOSS_SCALE = 1.25 / (B * HW * C)


def _vq_body(x_ref, w_ref, q_ref, idx_ref, loss_ref, perp_ref, pres_ref):
    b = pl.program_id(0)
    t = pl.program_id(1)
    x = x_ref[0]
    w = w_ref[...]
    wsq = jnp.sum(w * w, axis=1, keepdims=True)
    xsq = jnp.sum(x * x, axis=0, keepdims=True)
    scores = (wsq + xsq) - 2.0 * jax.lax.dot_general(
        w, x, (((1,), (0,)), ((), ())), preferred_element_type=jnp.float32)
    m = jnp.min(scores, axis=0, keepdims=True)
    iota_e = jax.lax.broadcasted_iota(jnp.int32, (E, TN), 0)
    idx = jnp.min(jnp.where(scores == m, iota_e, E), axis=0, keepdims=True)
    idx_ref[0] = idx
    onehot = (iota_e == idx).astype(jnp.float32)
    q = jax.lax.dot_general(
        w, onehot, (((0,), (0,)), ((), ())), preferred_element_type=jnp.float32)
    q_ref[0] = q

    @pl.when(jnp.logical_and(b == 0, t == 0))
    def _init():
        loss_ref[...] = jnp.zeros((1, 1), jnp.float32)
        perp_ref[...] = jnp.zeros((1, 1), jnp.float32)

    loss_ref[...] += jnp.sum(m, axis=1, keepdims=True)

    contrib = jnp.max(onehot, axis=1, keepdims=True)

    @pl.when(t == 0)
    def _reset():
        pres_ref[...] = jnp.zeros_like(pres_ref)

    pres_ref[...] = jnp.maximum(pres_ref[...], contrib)

    @pl.when(t == NT - 1)
    def _batch_done():
        perp_ref[...] += jnp.sum(pres_ref[...], axis=0, keepdims=True)

    @pl.when(jnp.logical_and(b == B - 1, t == NT - 1))
    def _finish():
        loss_ref[...] *= LOSS_SCALE
        perp_ref[...] *= 1.0 / B


def kernel(inputs, weight):
    x3 = inputs.reshape(B, C, HW)
    q, idxr, loss, perp = pl.pallas_call(
        _vq_body,
        grid=(B, NT),
        in_specs=[
            pl.BlockSpec((1, C, TN), lambda b, t: (b, 0, t)),
            pl.BlockSpec((E, C), lambda b, t: (0, 0)),
        ],
        out_specs=[
            pl.BlockSpec((1, C, TN), lambda b, t: (b, 0, t)),
            pl.BlockSpec((1, 1, TN), lambda b, t: (b * NT + t, 0, 0)),
            pl.BlockSpec((1, 1), lambda b, t: (0, 0)),
            pl.BlockSpec((1, 1), lambda b, t: (0, 0)),
        ],
        out_shape=[
            jax.ShapeDtypeStruct((B, C, HW), jnp.float32),
            jax.ShapeDtypeStruct((B * NT, 1, TN), jnp.int32),
            jax.ShapeDtypeStruct((1, 1), jnp.float32),
            jax.ShapeDtypeStruct((1, 1), jnp.float32),
        ],
        scratch_shapes=[pltpu.VMEM((E, 1), jnp.float32)],
        compiler_params=pltpu.CompilerParams(
            dimension_semantics=("arbitrary", "arbitrary")),
    )(x3, weight)
    quantized_out = q.reshape(B, C, 64, 64)
    encoding_indices = idxr.reshape(B, HW)
    return (loss[0, 0], quantized_out, perp[0, 0], encoding_indices)


# speedup vs baseline: 3.3574x; 1.0165x over previous
import jax
import jax.numpy as jnp
from jax.experimental import pallas as pl
from jax.experimental.pallas import tpu as pltpu

B = 16
C = 512
E = 512
HW = 64 * 64
TN = 4096
NT = HW // TN
LOSS_SCALE = 1.25 / (B * HW * C)


def _vq_body(x_ref, w_ref, q_ref, idx_ref, loss_ref, perp_ref, pres_ref):
    b = pl.program_id(0)
    t = pl.program_id(1)
    x = x_ref[0]
    w = w_ref[...]
    wsq = jnp.sum(w * w, axis=1, keepdims=True)
    xsq = jnp.sum(x * x, axis=0, keepdims=True)
    scores = (wsq + xsq) - 2.0 * jax.lax.dot_general(
        w, x, (((1,), (0,)), ((), ())), preferred_element_type=jnp.float32)
    m = jnp.min(scores, axis=0, keepdims=True)
    iota_e = jax.lax.broadcasted_iota(jnp.int32, (E, TN), 0)
    idx = jnp.min(jnp.where(scores == m, iota_e, E), axis=0, keepdims=True)
    idx_ref[0] = idx
    onehot = (iota_e == idx).astype(jnp.float32)
    q = jax.lax.dot_general(
        w, onehot, (((0,), (0,)), ((), ())), preferred_element_type=jnp.float32)
    q_ref[0] = q

    @pl.when(jnp.logical_and(b == 0, t == 0))
    def _init():
        loss_ref[...] = jnp.zeros((1, 1), jnp.float32)
        perp_ref[...] = jnp.zeros((1, 1), jnp.float32)

    loss_ref[...] += jnp.sum(m, axis=1, keepdims=True)

    contrib = jnp.max(onehot, axis=1, keepdims=True)

    @pl.when(t == 0)
    def _reset():
        pres_ref[...] = jnp.zeros_like(pres_ref)

    pres_ref[...] = jnp.maximum(pres_ref[...], contrib)

    @pl.when(t == NT - 1)
    def _batch_done():
        perp_ref[...] += jnp.sum(pres_ref[...], axis=0, keepdims=True)

    @pl.when(jnp.logical_and(b == B - 1, t == NT - 1))
    def _finish():
        loss_ref[...] *= LOSS_SCALE
        perp_ref[...] *= 1.0 / B


def kernel(inputs, weight):
    x3 = inputs.reshape(B, C, HW)
    q, idxr, loss, perp = pl.pallas_call(
        _vq_body,
        grid=(B, NT),
        in_specs=[
            pl.BlockSpec((1, C, TN), lambda b, t: (b, 0, t)),
            pl.BlockSpec((E, C), lambda b, t: (0, 0)),
        ],
        out_specs=[
            pl.BlockSpec((1, C, TN), lambda b, t: (b, 0, t)),
            pl.BlockSpec((1, 1, TN), lambda b, t: (b * NT + t, 0, 0)),
            pl.BlockSpec((1, 1), lambda b, t: (0, 0)),
            pl.BlockSpec((1, 1), lambda b, t: (0, 0)),
        ],
        out_shape=[
            jax.ShapeDtypeStruct((B, C, HW), jnp.float32),
            jax.ShapeDtypeStruct((B * NT, 1, TN), jnp.int32),
            jax.ShapeDtypeStruct((1, 1), jnp.float32),
            jax.ShapeDtypeStruct((1, 1), jnp.float32),
        ],
        scratch_shapes=[pltpu.VMEM((E, 1), jnp.float32)],
        compiler_params=pltpu.CompilerParams(
            dimension_semantics=("arbitrary", "arbitrary")),
    )(x3, weight)
    quantized_out = q.reshape(B, C, 64, 64)
    encoding_indices = idxr.reshape(B, HW)
    return (loss[0, 0], quantized_out, perp[0, 0], encoding_indices)
